# baseline (speedup 1.0000x reference)
import functools

import jax
import jax.numpy as jnp
from jax import lax
from jax.experimental import pallas as pl
from jax.experimental.pallas import tpu as pltpu
from jax.experimental.pallas import tpu_sc as plsc


_N_ROWS = 16384
_N_COLS = 128
_NW = 32
_CHUNK = _N_ROWS // _NW


_mesh = plsc.VectorSubcoreMesh(core_axis_name="c", subcore_axis_name="s")


@functools.partial(
    pl.kernel,
    mesh=_mesh,
    out_type=jax.ShapeDtypeStruct((_N_ROWS, _N_COLS), jnp.float32),
    scratch_types=[
        pltpu.VMEM((_CHUNK, _N_COLS), jnp.float32),
        pltpu.VMEM((32,), jnp.float32),
        pltpu.SemaphoreType.DMA,
    ],
)
def _sc_copy(x_hbm, y_hbm, out_hbm, buf, yv32, sem):
    wid = lax.axis_index("s") * 2 + lax.axis_index("c")
    base = wid * _CHUNK
    cp = pltpu.make_async_copy(x_hbm.at[pl.ds(base, _CHUNK), :], buf, sem)
    cp.start()

    @pl.when(wid == 0)
    def _stage_y():
        pltpu.sync_copy(y_hbm.at[0, pl.ds(0, 16)], yv32.at[pl.ds(8, 16)])

    cp.wait()

    @pl.when(wid == 0)
    def _patch():
        shifted = yv32[pl.ds(7, 16)]
        row16 = buf[0, pl.ds(0, 16)]
        lane = lax.iota(jnp.int32, 16)
        buf[0, pl.ds(0, 16)] = jnp.where(lane == 3, shifted, row16)

    pltpu.sync_copy(buf, out_hbm.at[pl.ds(base, _CHUNK), :])


def kernel(x, y):
    return _sc_copy(x, y)


# speedup vs baseline: 4.2170x; 4.2170x over previous
import jax
import jax.numpy as jnp
from jax.experimental import pallas as pl
from jax.experimental.pallas import tpu as pltpu


_CHUNK_ROWS = 8192
_N_CHUNKS = 2
_NBUF = 2


def _body(x_ref, y_ref, o_ref, bufs, ybuf, in_sems, out_sems, ysem):
    y_cp = pltpu.make_async_copy(y_ref.at[pl.ds(0, 8), :], ybuf, ysem)
    y_cp.start()

    def in_copy(c):
        b = c % _NBUF
        return pltpu.make_async_copy(
            x_ref.at[pl.ds(c * _CHUNK_ROWS, _CHUNK_ROWS), :],
            bufs.at[b], in_sems.at[b])

    def out_copy(c):
        b = c % _NBUF
        return pltpu.make_async_copy(
            bufs.at[b],
            o_ref.at[pl.ds(c * _CHUNK_ROWS, _CHUNK_ROWS), :],
            out_sems.at[b])

    for c in range(_NBUF):
        in_copy(c).start()
    y_cp.wait()

    for c in range(_N_CHUNKS):
        in_copy(c).wait()
        if c == 0:
            col = jax.lax.broadcasted_iota(jnp.int32, (1, 128), 1)
            bufs[0, 0:1, :] = jnp.where(col == 3, ybuf[0, 2], bufs[0, 0:1, :])
        out_copy(c).start()
        if c + _NBUF < _N_CHUNKS:
            out_copy(c).wait()
            in_copy(c + _NBUF).start()

    for c in range(_N_CHUNKS - _NBUF, _N_CHUNKS):
        out_copy(c).wait()


def kernel(x, y):
    n_rows, n_cols = x.shape
    return pl.pallas_call(
        _body,
        in_specs=[
            pl.BlockSpec(memory_space=pltpu.MemorySpace.HBM),
            pl.BlockSpec(memory_space=pltpu.MemorySpace.HBM),
        ],
        out_specs=pl.BlockSpec(memory_space=pltpu.MemorySpace.HBM),
        out_shape=jax.ShapeDtypeStruct(x.shape, x.dtype),
        scratch_shapes=[
            pltpu.VMEM((_NBUF, _CHUNK_ROWS, n_cols), x.dtype),
            pltpu.VMEM((8, n_cols), y.dtype),
            pltpu.SemaphoreType.DMA((_NBUF,)),
            pltpu.SemaphoreType.DMA((_NBUF,)),
            pltpu.SemaphoreType.DMA,
        ],
    )(x, y)
